# baseline (speedup 1.0000x reference)
import functools

import jax
import jax.numpy as jnp
from jax import lax
from jax.experimental import pallas as pl
from jax.experimental.pallas import tpu as pltpu
from jax.experimental.pallas import tpu_sc as plsc

NUM_NEXT = 16384
CTX_LEN = 200
CTX_PAD = 256
FEAT = 32

_NC = 2
_NS = 16
_NW = _NC * _NS
_NEXT_PER_W = NUM_NEXT // _NW
_CTX_PER_W = CTX_PAD // _NW


def _sc_gather_body(tt, at, rt, nti, nai, nri, cti, cai, cri,
                    nt_out, na_out, nr_out, ct_out, ca_out, cr_out,
                    in0, in1, in2, rn0, rn1, rn2,
                    ic0, ic1, ic2, rc0, rc1, rc2, sems):
    wid = lax.axis_index("s") * _NC + lax.axis_index("c")
    nbase = wid * _NEXT_PER_W
    cbase = wid * _CTX_PER_W
    tables = (tt, at, rt)
    nidx = (nti, nai, nri)
    cidx = (cti, cai, cri)
    nout = (nt_out, na_out, nr_out)
    cout = (ct_out, ca_out, cr_out)
    idx_n = (in0, in1, in2)
    rows_n = (rn0, rn1, rn2)
    idx_c = (ic0, ic1, ic2)
    rows_c = (rc0, rc1, rc2)
    copies = []
    for k in range(3):
        pltpu.sync_copy(nidx[k].at[pl.ds(nbase, _NEXT_PER_W)], idx_n[k])
        pltpu.sync_copy(cidx[k].at[pl.ds(cbase, _CTX_PER_W)], idx_c[k])
        copies.append(pltpu.async_copy(tables[k].at[idx_n[k]], rows_n[k],
                                       sems.at[2 * k]))
        copies.append(pltpu.async_copy(tables[k].at[idx_c[k]], rows_c[k],
                                       sems.at[2 * k + 1]))
    for c in copies:
        c.wait()
    for k in range(3):
        pltpu.sync_copy(rows_n[k], nout[k].at[pl.ds(nbase, _NEXT_PER_W)])
        pltpu.sync_copy(rows_c[k], cout[k].at[pl.ds(cbase, _CTX_PER_W)])


def _sc_gather(tt, at, rt, nti, nai, nri, cti, cai, cri):
    mesh = plsc.VectorSubcoreMesh(core_axis_name="c", subcore_axis_name="s")
    f = pl.kernel(
        _sc_gather_body,
        out_type=(
            jax.ShapeDtypeStruct((NUM_NEXT, FEAT), jnp.float32),
            jax.ShapeDtypeStruct((NUM_NEXT, FEAT), jnp.float32),
            jax.ShapeDtypeStruct((NUM_NEXT, FEAT), jnp.float32),
            jax.ShapeDtypeStruct((CTX_PAD, FEAT), jnp.float32),
            jax.ShapeDtypeStruct((CTX_PAD, FEAT), jnp.float32),
            jax.ShapeDtypeStruct((CTX_PAD, FEAT), jnp.float32),
        ),
        mesh=mesh,
        compiler_params=pltpu.CompilerParams(use_tc_tiling_on_sc=False),
        scratch_types=(
            [pltpu.VMEM((_NEXT_PER_W,), jnp.int32)] * 3
            + [pltpu.VMEM((_NEXT_PER_W, FEAT), jnp.float32)] * 3
            + [pltpu.VMEM((_CTX_PER_W,), jnp.int32)] * 3
            + [pltpu.VMEM((_CTX_PER_W, FEAT), jnp.float32)] * 3
            + [pltpu.SemaphoreType.DMA((6,))]
        ),
    )
    return f(tt, at, rt, nti, nai, nri, cti, cai, cri)


def _tc_affinity_body(nt, na, nr, ct, ca, cr, out):
    acc = jnp.dot(nt[...], ct[...].T, preferred_element_type=jnp.float32)
    acc += jnp.dot(na[...], ca[...].T, preferred_element_type=jnp.float32)
    acc += jnp.dot(nr[...], cr[...].T, preferred_element_type=jnp.float32)
    col = lax.broadcasted_iota(jnp.int32, acc.shape, 1)
    acc = jnp.where(col < CTX_LEN, acc, -jnp.inf)
    out[...] = jnp.max(acc, axis=1)


def _tc_affinity(nt, na, nr, ct, ca, cr, block=2048, interpret=False):
    grid = (NUM_NEXT // block,)
    nspec = pl.BlockSpec((block, FEAT), lambda i: (i, 0))
    cspec = pl.BlockSpec((CTX_PAD, FEAT), lambda i: (0, 0))
    return pl.pallas_call(
        _tc_affinity_body,
        grid=grid,
        in_specs=[nspec, nspec, nspec, cspec, cspec, cspec],
        out_specs=pl.BlockSpec((block,), lambda i: (i,)),
        out_shape=jax.ShapeDtypeStruct((NUM_NEXT,), jnp.float32),
        interpret=interpret,
    )(nt, na, nr, ct, ca, cr)


def kernel(track_context, album_context, artist_context,
           next_track, next_album, next_artist,
           track_table, album_table, artist_table):
    pad = CTX_PAD - CTX_LEN
    cti = jnp.pad(track_context, (0, pad))
    cai = jnp.pad(album_context, (0, pad))
    cri = jnp.pad(artist_context, (0, pad))
    nt, na, nr, ct, ca, cr = _sc_gather(
        track_table, album_table, artist_table,
        next_track, next_album, next_artist, cti, cai, cri)
    return _tc_affinity(nt, na, nr, ct, ca, cr)
